# baseline (speedup 1.0000x reference)
import functools

import jax
import jax.numpy as jnp
from jax import lax
from jax.experimental import pallas as pl
from jax.experimental.pallas import tpu as pltpu
from jax.experimental.pallas import tpu_sc as plsc

B, C, H, W = 8, 256, 128, 128
HW = H * W
NPIX = B * HW
NCLS = 21
NW = 32
CSC = 64
CT = CSC // 4
NBINS = 24
BANKT = NBINS * CT
SHIFTB = 14 - (CT.bit_length() - 1)
CNTB = 128
L = 16
SR = 4096
NROUND = HW // SR
NBUF = 4
PB = 2048
NPB = HW // PB
CHB = 64
NCB = (C - CSC) // CHB
CB0 = CSC // CHB
NBTC = 32
MOM = 0.99


def _seg_sums_sc(feats_flat, seg, prob, ig):
  mesh = plsc.VectorSubcoreMesh(core_axis_name="c", subcore_axis_name="s")

  @functools.partial(
      pl.kernel,
      mesh=mesh,
      compiler_params=pltpu.CompilerParams(needs_layout_passes=False),
      out_type=[
          jax.ShapeDtypeStruct((NW, BANKT), jnp.float32),
          jax.ShapeDtypeStruct((NW, CNTB), jnp.float32),
      ],
      scratch_types=[
          pltpu.VMEM((SR,), jnp.int32),
          pltpu.VMEM((SR,), jnp.float32),
          pltpu.VMEM((SR,), jnp.int32),
          pltpu.VMEM((HW + 2 * L,), jnp.int32),
          pltpu.VMEM((L * BANKT,), jnp.float32),
          pltpu.VMEM((L * CNTB,), jnp.float32),
          pltpu.VMEM((NBUF * HW,), jnp.float32),
          pltpu.SemaphoreType.DMA,
          pltpu.SemaphoreType.DMA,
          pltpu.SemaphoreType.DMA,
          pltpu.SemaphoreType.DMA,
          pltpu.SemaphoreType.DMA,
      ],
  )
  def k(feats_hbm, seg_hbm, prob_hbm, ig_hbm, out_sums, out_cnt,
        seg_v, prob_v, ig_v, pk_v, acc_v, cnt_v, fb,
        sem0, sem1, sem2, sem3, sem_in):
    wid = lax.axis_index("s") * 2 + lax.axis_index("c")
    b = wid // 4
    cq = wid % 4
    lane = lax.iota(jnp.int32, L)
    zf = jnp.zeros((L,), jnp.float32)
    sems = [sem0, sem1, sem2, sem3]

    def feat_copy(c, u, sem):
      return pltpu.make_async_copy(
          feats_hbm.at[pl.ds((b * C + cq * CT + c) * HW, HW)],
          fb.at[pl.ds(u * HW, HW)], sem)

    for u in range(NBUF - 1):
      feat_copy(u, u, sems[u]).start()

    def zero_acc(i, carry):
      acc_v[pl.ds(i * L, L)] = zf
      return carry

    lax.fori_loop(0, L * BANKT // L, zero_acc, jnp.int32(0))

    def zero_cnt(i, carry):
      cnt_v[pl.ds(i * L, L)] = zf
      return carry

    lax.fori_loop(0, L * CNTB // L, zero_cnt, jnp.int32(0))

    ones = jnp.ones((L,), jnp.float32)
    count = jnp.int32(0)
    for r in range(NROUND):
      base = b * HW + r * SR
      pltpu.make_async_copy(seg_hbm.at[pl.ds(base, SR)], seg_v, sem_in).start()
      pltpu.make_async_copy(prob_hbm.at[pl.ds(base, SR)], prob_v, sem_in).start()
      pltpu.make_async_copy(ig_hbm.at[pl.ds(base, SR)], ig_v, sem_in).start()
      pltpu.make_async_copy(seg_hbm.at[pl.ds(base, SR)], seg_v, sem_in).wait()
      pltpu.make_async_copy(prob_hbm.at[pl.ds(base, SR)], prob_v, sem_in).wait()
      pltpu.make_async_copy(ig_hbm.at[pl.ds(base, SR)], ig_v, sem_in).wait()

      def compact(i, cn, r=r):
        s = seg_v[pl.ds(i * L, L)]
        pr = prob_v[pl.ds(i * L, L)]
        im = ig_v[pl.ds(i * L, L)]
        valid = (pr > 0.95) & (im != 255) & (s >= 0) & (s < NCLS)
        binv = jnp.where(valid, s, NCLS)
        plsc.addupdate_scatter(cnt_v, [lane * CNTB + binv], ones)
        packed = binv * HW + (lane + (r * SR + i * L))
        plsc.store_compressed(pk_v.at[pl.ds(cn, L)], packed, mask=valid)
        return cn + jnp.sum(valid.astype(jnp.int32))

      count = lax.fori_loop(0, SR // L, compact, count)

    dead = jnp.full((L,), NCLS * HW, jnp.int32)
    pk_v[pl.ds(count, L)] = dead
    pk_v[pl.ds(count + L, L)] = dead
    nvec2 = (count + 2 * L - 1) // (2 * L)

    lane_bank = lane * BANKT

    def gather_chunk(c, u):
      base_v = lane_bank + c
      buf = fb.at[pl.ds(u * HW, HW)]

      def g_body(j, carry):
        for h in range(2):
          w = pk_v[pl.ds(j * 2 * L + h * L, L)]
          pix = w & jnp.int32(HW - 1)
          binoff = lax.shift_right_logical(w & jnp.int32(~(HW - 1)), SHIFTB)
          val = plsc.load_gather(buf, [pix])
          plsc.addupdate_scatter(acc_v, [base_v + binoff], val)
        return carry

      lax.fori_loop(0, nvec2, g_body, jnp.int32(0))

    def ch_body(jn, carry):
      for u in range(NBUF):
        c = NBUF * jn + u
        feat_copy(c, u, sems[u]).wait()
        gather_chunk(c, u)

        @pl.when(c + NBUF < CT)
        def _():
          feat_copy(c + NBUF, u, sems[u]).start()

      return carry

    feat_copy(NBUF - 1, NBUF - 1, sems[NBUF - 1]).start()
    lax.fori_loop(0, CT // NBUF, ch_body, jnp.int32(0))

    def red_sums(j, carry):
      v = acc_v[pl.ds(j * L, L)]
      for l in range(1, L):
        v = v + acc_v[pl.ds(l * BANKT + j * L, L)]
      acc_v[pl.ds(j * L, L)] = v
      return carry

    lax.fori_loop(0, BANKT // L, red_sums, jnp.int32(0))

    def red_cnt(j, carry):
      v = cnt_v[pl.ds(j * L, L)]
      for l in range(1, L):
        v = v + cnt_v[pl.ds(l * CNTB + j * L, L)]
      cnt_v[pl.ds(j * L, L)] = v
      return carry

    lax.fori_loop(0, CNTB // L, red_cnt, jnp.int32(0))

    @pl.when(cq != 0)
    def _():
      def rez(j, carry):
        cnt_v[pl.ds(j * L, L)] = zf
        return carry

      lax.fori_loop(0, CNTB // L, rez, jnp.int32(0))

    pltpu.sync_copy(acc_v.at[pl.ds(0, BANKT)], out_sums.at[wid])
    pltpu.sync_copy(cnt_v.at[pl.ds(0, CNTB)], out_cnt.at[wid])

  return k(feats_flat, seg, prob, ig)


def _dense_body(f_ref, s_ref, p_ref, i_ref, o_ref):
  b = pl.program_id(1)
  pb = pl.program_id(2)

  @pl.when((b == 0) & (pb == 0))
  def _():
    o_ref[...] = jnp.zeros_like(o_ref)

  f = f_ref[...].reshape(CHB, PB)
  s = s_ref[...].reshape(1, PB)
  pr = p_ref[...].reshape(1, PB)
  ig = i_ref[...].reshape(1, PB)
  valid = (pr > 0.95) & (ig != 255)
  kidx = lax.broadcasted_iota(jnp.int32, (NBTC, PB), 0)
  oneh = ((kidx == s) & valid).astype(jnp.float32)
  acc = lax.dot_general(oneh, f, (((1,), (1,)), ((), ())),
                        preferred_element_type=jnp.float32)
  o_ref[...] += acc.reshape(1, NBTC, CHB)


def _dense_sums_tc(feats4, seg3, prob3, ig3):
  return pl.pallas_call(
      _dense_body,
      grid=(NCB, B, NPB),
      in_specs=[
          pl.BlockSpec((1, 1, CHB, PB), lambda cb, b, pb: (b, cb + CB0, 0, pb)),
          pl.BlockSpec((1, 1, PB), lambda cb, b, pb: (b * NPB + pb, 0, 0)),
          pl.BlockSpec((1, 1, PB), lambda cb, b, pb: (b * NPB + pb, 0, 0)),
          pl.BlockSpec((1, 1, PB), lambda cb, b, pb: (b * NPB + pb, 0, 0)),
      ],
      out_specs=pl.BlockSpec((1, NBTC, CHB), lambda cb, b, pb: (cb, 0, 0)),
      out_shape=jax.ShapeDtypeStruct((NCB, NBTC, CHB), jnp.float32),
  )(feats4, seg3, prob3, ig3)


def _combine_body(ssc_ref, stc_ref, c_ref, b_ref, o_ref):
  ssc = jnp.sum(ssc_ref[...], axis=0)
  s = jnp.concatenate([ssc[:NCLS], stc_ref[...][:NCLS]], axis=1)
  cn = jnp.sum(c_ref[...], axis=1, keepdims=True)
  c21 = cn[:NCLS]
  mean = s / jnp.maximum(c21, 1.0)
  present = c21 > 0.0
  row = b_ref[...]
  nz = jnp.sum((row == 0.0).astype(jnp.float32), axis=1, keepdims=True)
  is_zero = nz == float(C)
  do_copy = present & is_zero
  idx = lax.broadcasted_iota(jnp.int32, (NCLS, 1), 0)
  first = jnp.min(jnp.where(do_copy, idx, jnp.int32(2**30)))
  need = idx <= first
  do_mom = present & (~is_zero) & need
  mom_row = MOM * row + (1.0 - MOM) * mean
  o_ref[...] = jnp.where(do_copy, mean, jnp.where(do_mom, mom_row, row))


def _combine_tc(ssc3, stc, cnt_t, bank2):
  return pl.pallas_call(
      _combine_body,
      out_shape=jax.ShapeDtypeStruct((NCLS, C), jnp.float32),
  )(ssc3, stc, cnt_t, bank2)


def kernel(features, probablity_weak, memory_bank, segmentation, ignore_mask):
  feats_flat = features.reshape(B * C * HW)
  feats4 = features.reshape(B, C // CHB, CHB, HW)
  seg = segmentation.reshape(NPIX)
  prob = probablity_weak.reshape(NPIX)
  ig = ignore_mask.reshape(NPIX)
  seg3 = seg.reshape(B * NPB, 1, PB)
  prob3 = prob.reshape(B * NPB, 1, PB)
  ig3 = ig.reshape(B * NPB, 1, PB)
  sums, cnts = _seg_sums_sc(feats_flat, seg, prob, ig)
  stc = _dense_sums_tc(feats4, seg3, prob3, ig3)
  stc = stc.transpose(1, 0, 2).reshape(NBTC, C - CSC)
  ssc3 = sums.reshape(B, 4, NBINS, CT).transpose(0, 2, 1, 3).reshape(B, NBINS, CSC)
  out = _combine_tc(ssc3, stc, cnts.T, memory_bank.reshape(NCLS, C))
  return out.reshape(NCLS, 1, C)


# speedup vs baseline: 2.3816x; 2.3816x over previous
import functools

import jax
import jax.numpy as jnp
from jax import lax
from jax.experimental import pallas as pl
from jax.experimental.pallas import tpu as pltpu
from jax.experimental.pallas import tpu_sc as plsc

B, C, H, W = 8, 256, 128, 128
HW = H * W
NPIX = B * HW
NCLS = 21
NBINS = NCLS + 1
NW = 32
CT = C // 4
BANKT = NBINS * CT
SHIFTB = 14 - (CT.bit_length() - 1)
CNTB = 128
L = 16
SR = 2048
NROUND = HW // SR
NBUF = 4
PKH = HW // 2 + 2 * L
MOM = 0.99


def _seg_sums_sc(feats_flat, seg, prob, ig):
  mesh = plsc.VectorSubcoreMesh(core_axis_name="c", subcore_axis_name="s")

  @functools.partial(
      pl.kernel,
      mesh=mesh,
      compiler_params=pltpu.CompilerParams(needs_layout_passes=False),
      out_type=[
          jax.ShapeDtypeStruct((NW, BANKT), jnp.float32),
          jax.ShapeDtypeStruct((NW, CNTB), jnp.float32),
      ],
      scratch_types=[
          pltpu.VMEM((2 * SR,), jnp.int32),
          pltpu.VMEM((2 * SR,), jnp.float32),
          pltpu.VMEM((2 * SR,), jnp.int32),
          pltpu.VMEM((PKH,), jnp.int32),
          pltpu.VMEM((PKH,), jnp.int32),
          pltpu.VMEM((L * BANKT,), jnp.float32),
          pltpu.VMEM((L * CNTB,), jnp.float32),
          pltpu.VMEM((NBUF * HW,), jnp.float32),
          pltpu.SemaphoreType.DMA,
          pltpu.SemaphoreType.DMA,
          pltpu.SemaphoreType.DMA,
          pltpu.SemaphoreType.DMA,
          pltpu.SemaphoreType.DMA,
          pltpu.SemaphoreType.DMA,
      ],
  )
  def k(feats_hbm, seg_hbm, prob_hbm, ig_hbm, out_sums, out_cnt,
        seg_v, prob_v, ig_v, pkA, pkB, acc_v, cnt_v, fb,
        sem0, sem1, sem2, sem3, sem_in0, sem_in1):
    wid = lax.axis_index("s") * 2 + lax.axis_index("c")
    b = wid // 4
    cq = wid % 4
    lane = lax.iota(jnp.int32, L)
    zf = jnp.zeros((L,), jnp.float32)
    sems = [sem0, sem1, sem2, sem3]

    def feat_copy(c, u, sem):
      return pltpu.make_async_copy(
          feats_hbm.at[pl.ds((b * C + cq * CT + c) * HW, HW)],
          fb.at[pl.ds(u * HW, HW)], sem)

    for u in range(NBUF - 1):
      feat_copy(u, u, sems[u]).start()

    def zero_acc(i, carry):
      acc_v[pl.ds(i * L, L)] = zf
      return carry

    lax.fori_loop(0, L * BANKT // L, zero_acc, jnp.int32(0))

    def zero_cnt(i, carry):
      cnt_v[pl.ds(i * L, L)] = zf
      return carry

    lax.fori_loop(0, L * CNTB // L, zero_cnt, jnp.int32(0))

    ones = jnp.ones((L,), jnp.float32)
    sems_in = [sem_in0, sem_in1]

    def stage_start(r):
      st = r % 2
      base = b * HW + r * SR
      sem = sems_in[st]
      pltpu.make_async_copy(
          seg_hbm.at[pl.ds(base, SR)], seg_v.at[pl.ds(st * SR, SR)], sem).start()
      pltpu.make_async_copy(
          prob_hbm.at[pl.ds(base, SR)], prob_v.at[pl.ds(st * SR, SR)], sem).start()
      pltpu.make_async_copy(
          ig_hbm.at[pl.ds(base, SR)], ig_v.at[pl.ds(st * SR, SR)], sem).start()

    def stage_wait(r):
      st = r % 2
      base = b * HW + r * SR
      sem = sems_in[st]
      pltpu.make_async_copy(
          seg_hbm.at[pl.ds(base, SR)], seg_v.at[pl.ds(st * SR, SR)], sem).wait()
      pltpu.make_async_copy(
          prob_hbm.at[pl.ds(base, SR)], prob_v.at[pl.ds(st * SR, SR)], sem).wait()
      pltpu.make_async_copy(
          ig_hbm.at[pl.ds(base, SR)], ig_v.at[pl.ds(st * SR, SR)], sem).wait()

    cntA = jnp.int32(0)
    cntB = jnp.int32(0)
    stage_start(0)
    for r in range(NROUND):
      if r + 1 < NROUND:
        stage_start(r + 1)
      stage_wait(r)
      st = r % 2

      def compact(i, cns, r=r, st=st):
        cnA, cnB = cns
        for h in range(2):
          off = st * SR + i * 2 * L + h * L
          s = seg_v[pl.ds(off, L)]
          pr = prob_v[pl.ds(off, L)]
          im = ig_v[pl.ds(off, L)]
          valid = (pr > 0.95) & (im != 255) & (s >= 0) & (s < NCLS)
          binv = jnp.where(valid, s, NCLS)
          plsc.addupdate_scatter(cnt_v, [lane * CNTB + binv], ones)
          packed = binv * HW + (lane + (r * SR + i * 2 * L + h * L))
          if h == 0:
            plsc.store_compressed(pkA.at[pl.ds(cnA, L)], packed, mask=valid)
            cnA = cnA + jnp.sum(valid.astype(jnp.int32))
          else:
            plsc.store_compressed(pkB.at[pl.ds(cnB, L)], packed, mask=valid)
            cnB = cnB + jnp.sum(valid.astype(jnp.int32))
        return (cnA, cnB)

      cntA, cntB = lax.fori_loop(0, SR // (2 * L), compact, (cntA, cntB))

    dead = jnp.full((L,), NCLS * HW, jnp.int32)
    pkA[pl.ds(cntA, L)] = dead
    pkA[pl.ds(cntA + L, L)] = dead
    pkB[pl.ds(cntB, L)] = dead
    pkB[pl.ds(cntB + L, L)] = dead
    nvecA = (cntA + 2 * L - 1) // (2 * L)
    nvecB = (cntB + 2 * L - 1) // (2 * L)

    lane_bank = lane * BANKT

    def gather_chunk(c, u):
      base_v = lane_bank + c
      buf = fb.at[pl.ds(u * HW, HW)]

      def make_body(pk):
        def g_body(j, carry):
          for h in range(2):
            w = pk[pl.ds(j * 2 * L + h * L, L)]
            pix = w & jnp.int32(HW - 1)
            binoff = lax.shift_right_logical(w & jnp.int32(~(HW - 1)), SHIFTB)
            val = plsc.load_gather(buf, [pix])
            plsc.addupdate_scatter(acc_v, [base_v + binoff], val)
          return carry

        return g_body

      lax.fori_loop(0, nvecA, make_body(pkA), jnp.int32(0))
      lax.fori_loop(0, nvecB, make_body(pkB), jnp.int32(0))

    def ch_body(jn, carry):
      for u in range(NBUF):
        c = NBUF * jn + u
        feat_copy(c, u, sems[u]).wait()
        gather_chunk(c, u)

        @pl.when(c + NBUF < CT)
        def _():
          feat_copy(c + NBUF, u, sems[u]).start()

      return carry

    feat_copy(NBUF - 1, NBUF - 1, sems[NBUF - 1]).start()
    lax.fori_loop(0, CT // NBUF, ch_body, jnp.int32(0))

    def red_sums(j, carry):
      v = acc_v[pl.ds(j * L, L)]
      for l in range(1, L):
        v = v + acc_v[pl.ds(l * BANKT + j * L, L)]
      acc_v[pl.ds(j * L, L)] = v
      return carry

    lax.fori_loop(0, BANKT // L, red_sums, jnp.int32(0))

    def red_cnt(j, carry):
      v = cnt_v[pl.ds(j * L, L)]
      for l in range(1, L):
        v = v + cnt_v[pl.ds(l * CNTB + j * L, L)]
      cnt_v[pl.ds(j * L, L)] = v
      return carry

    lax.fori_loop(0, CNTB // L, red_cnt, jnp.int32(0))

    @pl.when(cq != 0)
    def _():
      def rez(j, carry):
        cnt_v[pl.ds(j * L, L)] = zf
        return carry

      lax.fori_loop(0, CNTB // L, rez, jnp.int32(0))

    pltpu.sync_copy(acc_v.at[pl.ds(0, BANKT)], out_sums.at[wid])
    pltpu.sync_copy(cnt_v.at[pl.ds(0, CNTB)], out_cnt.at[wid])

  return k(feats_flat, seg, prob, ig)


def _combine_body(s_ref, c_ref, b_ref, o_ref):
  s = jnp.sum(s_ref[...], axis=0)
  cn = jnp.sum(c_ref[...], axis=1, keepdims=True)
  s21 = s[:NCLS]
  c21 = cn[:NCLS]
  mean = s21 / jnp.maximum(c21, 1.0)
  present = c21 > 0.0
  row = b_ref[...]
  nz = jnp.sum((row == 0.0).astype(jnp.float32), axis=1, keepdims=True)
  is_zero = nz == float(C)
  do_copy = present & is_zero
  idx = lax.broadcasted_iota(jnp.int32, (NCLS, 1), 0)
  first = jnp.min(jnp.where(do_copy, idx, jnp.int32(2**30)))
  need = idx <= first
  do_mom = present & (~is_zero) & need
  mom_row = MOM * row + (1.0 - MOM) * mean
  o_ref[...] = jnp.where(do_copy, mean, jnp.where(do_mom, mom_row, row))


def _combine_tc(sums3, cnt_t, bank2):
  return pl.pallas_call(
      _combine_body,
      out_shape=jax.ShapeDtypeStruct((NCLS, C), jnp.float32),
  )(sums3, cnt_t, bank2)


def kernel(features, probablity_weak, memory_bank, segmentation, ignore_mask):
  feats_flat = features.reshape(B * C * HW)
  seg = segmentation.reshape(NPIX)
  prob = probablity_weak.reshape(NPIX)
  ig = ignore_mask.reshape(NPIX)
  sums, cnts = _seg_sums_sc(feats_flat, seg, prob, ig)
  sums_b = sums.reshape(B, 4, NBINS, CT).transpose(0, 2, 1, 3).reshape(B, NBINS, C)
  out = _combine_tc(sums_b, cnts.T, memory_bank.reshape(NCLS, C))
  return out.reshape(NCLS, 1, C)
